# baseline (speedup 1.0000x reference)
import functools

import jax
import jax.numpy as jnp
from jax import lax
from jax.experimental import pallas as pl
from jax.experimental.pallas import tpu as pltpu
from jax.experimental.pallas import tpu_sc as plsc

_IDX_CHUNK = 128


def _pair_gather(pair_ids_2d, pair_table):
    total_chunks, idx_chunk = pair_ids_2d.shape
    _, dim2 = pair_table.shape
    info = plsc.get_sparse_core_info()
    nw = info.num_cores * info.num_subcores
    n_chunks = total_chunks // nw
    rows_per_w = n_chunks * idx_chunk

    mesh = plsc.VectorSubcoreMesh(core_axis_name="c", subcore_axis_name="s")

    @functools.partial(
        pl.kernel,
        mesh=mesh,
        out_type=jax.ShapeDtypeStruct((total_chunks * idx_chunk, dim2), jnp.float32),
        scratch_types=[
            pltpu.VMEM((n_chunks, idx_chunk), jnp.int32),
            pltpu.VMEM((rows_per_w, dim2), jnp.float32),
            pltpu.SemaphoreType.DMA,
            pltpu.SemaphoreType.DMA,
        ],
    )
    def emb(idx_hbm, table_hbm, out_hbm, idx_v, rows_v, gsem, wsem):
        wid = lax.axis_index("s") * info.num_cores + lax.axis_index("c")
        pltpu.sync_copy(idx_hbm.at[pl.ds(wid * n_chunks, n_chunks)], idx_v)
        gathers = [
            pltpu.async_copy(
                table_hbm.at[idx_v.at[k]],
                rows_v.at[pl.ds(k * idx_chunk, idx_chunk)],
                gsem,
            )
            for k in range(n_chunks)
        ]
        writes = []
        for k in range(n_chunks):
            gathers[k].wait()
            writes.append(
                pltpu.async_copy(
                    rows_v.at[pl.ds(k * idx_chunk, idx_chunk)],
                    out_hbm.at[pl.ds(wid * rows_per_w + k * idx_chunk, idx_chunk)],
                    wsem,
                )
            )
        for w in writes:
            w.wait()

    return emb(pair_ids_2d, pair_table)


def kernel(emotion_ids, embedding_weight):
    ids = emotion_ids.astype(jnp.int32)
    table = embedding_weight.astype(jnp.float32)
    vocab, dim = table.shape
    batch, = ids.shape

    left = jnp.repeat(table, vocab, axis=0)
    right = jnp.tile(table, (vocab, 1))
    pair_table = jnp.concatenate([left, right], axis=1)

    x = ids.reshape(batch // 128, 128)
    pair_ids_2d = x[:, 0::2] * vocab + x[:, 1::2]

    out2 = _pair_gather(pair_ids_2d, pair_table)
    return out2.reshape(batch, dim)


# speedup vs baseline: 1.2927x; 1.2927x over previous
import functools

import jax
import jax.numpy as jnp
from jax import lax
from jax.experimental import pallas as pl
from jax.experimental.pallas import tpu as pltpu
from jax.experimental.pallas import tpu_sc as plsc

_IDX_CHUNK = 128


def _pair_gather(pair_ids_2d, pair_table):
    total_chunks, idx_chunk = pair_ids_2d.shape
    _, dim2 = pair_table.shape
    info = plsc.get_sparse_core_info()
    nw = info.num_cores * info.num_subcores
    n_chunks = total_chunks // nw
    rows_per_w = n_chunks * idx_chunk

    mesh = plsc.VectorSubcoreMesh(core_axis_name="c", subcore_axis_name="s")

    @functools.partial(
        pl.kernel,
        mesh=mesh,
        out_type=jax.ShapeDtypeStruct((total_chunks * idx_chunk, dim2), jnp.float32),
        scratch_types=[
            pltpu.VMEM((n_chunks, idx_chunk), jnp.int32),
            pltpu.VMEM((rows_per_w, dim2), jnp.float32),
            pltpu.VMEM_SHARED((81, dim2), jnp.float32),
            pltpu.SemaphoreType.DMA,
            pltpu.SemaphoreType.DMA,
        ],
    )
    def emb(idx_hbm, table_hbm, out_hbm, idx_v, rows_v, table_s, gsem, wsem):
        cid = lax.axis_index("c")
        sid = lax.axis_index("s")
        wid = sid * info.num_cores + cid

        @pl.when(sid == 0)
        def _():
            pltpu.sync_copy(table_hbm, table_s)

        pltpu.sync_copy(idx_hbm.at[pl.ds(wid * n_chunks, n_chunks)], idx_v)
        plsc.subcore_barrier()
        gathers = [
            pltpu.async_copy(
                table_s.at[idx_v.at[k]],
                rows_v.at[pl.ds(k * idx_chunk, idx_chunk)],
                gsem,
            )
            for k in range(n_chunks)
        ]
        writes = []
        for k in range(n_chunks):
            gathers[k].wait()
            writes.append(
                pltpu.async_copy(
                    rows_v.at[pl.ds(k * idx_chunk, idx_chunk)],
                    out_hbm.at[pl.ds(wid * rows_per_w + k * idx_chunk, idx_chunk)],
                    wsem,
                )
            )
        for w in writes:
            w.wait()

    return emb(pair_ids_2d, pair_table)


def kernel(emotion_ids, embedding_weight):
    ids = emotion_ids.astype(jnp.int32)
    table = embedding_weight.astype(jnp.float32)
    vocab, dim = table.shape
    batch, = ids.shape

    left = jnp.repeat(table, vocab, axis=0)
    right = jnp.tile(table, (vocab, 1))
    pair_table = jnp.concatenate([left, right], axis=1)

    x = ids.reshape(batch // 256, 256)
    pair_ids_2d = x[:, 0::2] * vocab + x[:, 1::2]

    out2 = _pair_gather(pair_ids_2d, pair_table)
    return out2.reshape(batch, dim)


# speedup vs baseline: 1.3282x; 1.0275x over previous
import functools

import jax
import jax.numpy as jnp
from jax import lax
from jax.experimental import pallas as pl
from jax.experimental.pallas import tpu as pltpu
from jax.experimental.pallas import tpu_sc as plsc

_L = 16


def _vperm(v, idx):
    dn = lax.GatherDimensionNumbers(
        offset_dims=(), collapsed_slice_dims=(0,), start_index_map=(0,)
    )
    return lax.gather(
        v, idx[:, None], dn, (1,),
        mode=lax.GatherScatterMode.PROMISE_IN_BOUNDS,
    )


def _evens_odds(a, b):
    lane = lax.iota(jnp.int32, _L)
    perm_lo = (2 * lane) % _L
    perm_hi = (2 * lane + 1) % _L
    in_lo = lane < (_L // 2)
    ev = jnp.where(in_lo, _vperm(a, perm_lo), _vperm(b, perm_lo))
    od = jnp.where(in_lo, _vperm(a, perm_hi), _vperm(b, perm_hi))
    return ev, od


def _emb_lookup(ids2d, table):
    nrow, ncol = ids2d.shape
    vocab, dim = table.shape
    dim2 = 2 * dim
    info = plsc.get_sparse_core_info()
    nw = info.num_cores * info.num_subcores
    rows_per_w = nrow // nw
    n_chunks = rows_per_w // 2
    pairs_per_w = rows_per_w * ncol // 2

    mesh = plsc.VectorSubcoreMesh(core_axis_name="c", subcore_axis_name="s")

    @functools.partial(
        pl.kernel,
        mesh=mesh,
        out_type=jax.ShapeDtypeStruct((nrow * ncol // 2, dim2), jnp.float32),
        scratch_types=[
            pltpu.VMEM((rows_per_w, ncol), jnp.int32),
            pltpu.VMEM((n_chunks, ncol), jnp.int32),
            pltpu.VMEM((pairs_per_w, dim2), jnp.float32),
            pltpu.VMEM((vocab * vocab, dim2), jnp.float32),
            pltpu.VMEM((vocab, dim), jnp.float32),
            pltpu.VMEM_SHARED((vocab * vocab, dim2), jnp.float32),
            pltpu.SemaphoreType.DMA,
            pltpu.SemaphoreType.DMA,
        ],
    )
    def emb(
        ids_hbm,
        table_hbm,
        out_hbm,
        raw_v,
        idx_v,
        rows_v,
        pairs_v,
        table_v,
        table_s,
        gsem,
        wsem,
    ):
        cid = lax.axis_index("c")
        sid = lax.axis_index("s")
        wid = sid * info.num_cores + cid

        pltpu.sync_copy(ids_hbm.at[pl.ds(wid * rows_per_w, rows_per_w)], raw_v)

        @pl.when(sid == 0)
        def _():
            pltpu.sync_copy(table_hbm, table_v)
            rows = [
                [table_v[i, pl.ds(c * _L, _L)] for c in range(dim // _L)]
                for i in range(vocab)
            ]
            for i in range(vocab):
                for j in range(vocab):
                    p = i * vocab + j
                    for c in range(dim // _L):
                        pairs_v[p, pl.ds(c * _L, _L)] = rows[i][c]
                        pairs_v[p, pl.ds(dim + c * _L, _L)] = rows[j][c]
            pltpu.sync_copy(pairs_v, table_s)

        for n in range(pairs_per_w // _L):
            r, c0 = divmod(32 * n, ncol)
            a = raw_v[r, pl.ds(c0, _L)]
            b = raw_v[r, pl.ds(c0 + _L, _L)]
            ev, od = _evens_odds(a, b)
            pid = ev * vocab + od
            qr, qc = divmod(_L * n, ncol)
            idx_v[qr, pl.ds(qc, _L)] = pid

        plsc.subcore_barrier()

        gathers = [
            pltpu.async_copy(
                table_s.at[idx_v.at[k]],
                rows_v.at[pl.ds(k * ncol, ncol)],
                gsem,
            )
            for k in range(n_chunks)
        ]
        writes = []
        for k in range(n_chunks):
            gathers[k].wait()
            writes.append(
                pltpu.async_copy(
                    rows_v.at[pl.ds(k * ncol, ncol)],
                    out_hbm.at[pl.ds(wid * pairs_per_w + k * ncol, ncol)],
                    wsem,
                )
            )
        for w in writes:
            w.wait()

    return emb(ids2d, table)


def kernel(emotion_ids, embedding_weight):
    ids = emotion_ids.astype(jnp.int32)
    table = embedding_weight.astype(jnp.float32)
    vocab, dim = table.shape
    batch, = ids.shape

    ids2d = ids.reshape(batch // 128, 128)
    out2 = _emb_lookup(ids2d, table)
    return out2.reshape(batch, dim)


# speedup vs baseline: 1.3415x; 1.0100x over previous
import functools

import jax
import jax.numpy as jnp
from jax import lax
from jax.experimental import pallas as pl
from jax.experimental.pallas import tpu as pltpu
from jax.experimental.pallas import tpu_sc as plsc

_L = 16


def _vperm(v, idx):
    dn = lax.GatherDimensionNumbers(
        offset_dims=(), collapsed_slice_dims=(0,), start_index_map=(0,)
    )
    return lax.gather(
        v, idx[:, None], dn, (1,),
        mode=lax.GatherScatterMode.PROMISE_IN_BOUNDS,
    )


def _evens_odds(a, b):
    lane = lax.iota(jnp.int32, _L)
    perm_lo = (2 * lane) % _L
    perm_hi = (2 * lane + 1) % _L
    in_lo = lane < (_L // 2)
    ev = jnp.where(in_lo, _vperm(a, perm_lo), _vperm(b, perm_lo))
    od = jnp.where(in_lo, _vperm(a, perm_hi), _vperm(b, perm_hi))
    return ev, od


def _emb_lookup(ids2d, table):
    nrow, ncol = ids2d.shape
    vocab, dim = table.shape
    dim2 = 2 * dim
    info = plsc.get_sparse_core_info()
    nw = info.num_cores * info.num_subcores
    rows_per_w = nrow // nw
    n_chunks = rows_per_w // 2
    pairs_per_w = rows_per_w * ncol // 2

    mesh = plsc.VectorSubcoreMesh(core_axis_name="c", subcore_axis_name="s")

    @functools.partial(
        pl.kernel,
        mesh=mesh,
        out_type=jax.ShapeDtypeStruct((nrow * ncol // 2, dim2), jnp.float32),
        scratch_types=[
            pltpu.VMEM((rows_per_w, ncol), jnp.int32),
            pltpu.VMEM((n_chunks, ncol), jnp.int32),
            pltpu.VMEM((pairs_per_w, dim2), jnp.float32),
            pltpu.VMEM((vocab * vocab, dim2), jnp.float32),
            pltpu.VMEM((vocab, dim), jnp.float32),
            pltpu.VMEM_SHARED((vocab * vocab, dim2), jnp.float32),
            pltpu.SemaphoreType.DMA,
            pltpu.SemaphoreType.DMA,
        ],
    )
    def emb(
        ids_hbm,
        table_hbm,
        out_hbm,
        raw_v,
        idx_v,
        rows_v,
        pairs_v,
        table_v,
        table_s,
        gsem,
        wsem,
    ):
        cid = lax.axis_index("c")
        sid = lax.axis_index("s")
        wid = sid * info.num_cores + cid

        pltpu.sync_copy(ids_hbm.at[pl.ds(wid * rows_per_w, rows_per_w)], raw_v)

        @pl.when(sid == 0)
        def _():
            pltpu.sync_copy(table_hbm, table_v)
            rows = [
                [table_v[i, pl.ds(c * _L, _L)] for c in range(dim // _L)]
                for i in range(vocab)
            ]
            for i in range(vocab):
                for j in range(vocab):
                    p = i * vocab + j
                    for c in range(dim // _L):
                        pairs_v[p, pl.ds(c * _L, _L)] = rows[i][c]
                        pairs_v[p, pl.ds(dim + c * _L, _L)] = rows[j][c]
            pltpu.sync_copy(pairs_v, table_s)

        for n in range(pairs_per_w // _L):
            r, c0 = divmod(32 * n, ncol)
            a = raw_v[r, pl.ds(c0, _L)]
            b = raw_v[r, pl.ds(c0 + _L, _L)]
            ev, od = _evens_odds(a, b)
            pid = ev * vocab + od
            qr, qc = divmod(_L * n, ncol)
            idx_v[qr, pl.ds(qc, _L)] = pid

        plsc.subcore_barrier()

        half = ncol // 2
        gathers = [
            pltpu.async_copy(
                table_s.at[idx_v.at[k // 2, pl.ds((k % 2) * half, half)]],
                rows_v.at[pl.ds(k * half, half)],
                gsem,
            )
            for k in range(2 * n_chunks)
        ]
        writes = []
        for k in range(2 * n_chunks):
            gathers[k].wait()
            writes.append(
                pltpu.async_copy(
                    rows_v.at[pl.ds(k * half, half)],
                    out_hbm.at[pl.ds(wid * pairs_per_w + k * half, half)],
                    wsem,
                )
            )
        for w in writes:
            w.wait()

    return emb(ids2d, table)


def kernel(emotion_ids, embedding_weight):
    ids = emotion_ids.astype(jnp.int32)
    table = embedding_weight.astype(jnp.float32)
    vocab, dim = table.shape
    batch, = ids.shape

    ids2d = ids.reshape(batch // 128, 128)
    out2 = _emb_lookup(ids2d, table)
    return out2.reshape(batch, dim)
